# baseline (speedup 1.0000x reference)
import functools

import jax
import jax.numpy as jnp
import numpy as np
from jax import lax
from jax.experimental import pallas as pl
from jax.experimental.pallas import tpu as pltpu
from jax.experimental.pallas import tpu_sc as plsc

N_NODES = 10000
N_EDGES = 320000
D_IN = 128
D_H1 = 64
D_H2 = 32

NC = 2
NS = 16
NW = NC * NS
E_PER_W = N_EDGES // NW
BLK = 200
NB = E_PER_W // BLK
ROWS_PER_SUB = 632
N_PAD = ROWS_PER_SUB * NS
NSLOT = 4
LOOKAHEAD = 3

_MESH = plsc.VectorSubcoreMesh(core_axis_name="c", subcore_axis_name="s")

_F32 = jnp.float32

_ONES_BLK = np.ones((BLK, 16), np.float32)
NB_MAIN = (NB // NSLOT) * NSLOT

_Z64 = np.zeros((N_PAD, D_H1), np.float32)
_Z32 = np.zeros((N_PAD, D_H2), np.float32)
_Z16 = np.zeros((N_PAD, 16), np.float32)


def _sc_segsum(y, e_r):
    d = y.shape[1]
    zD = _Z64 if d == D_H1 else _Z32
    out_type = jax.ShapeDtypeStruct((NC, N_PAD, d), _F32)
    scratch = [
        pltpu.VMEM((NB, BLK), jnp.int32),
        pltpu.VMEM((NB, BLK), jnp.int32),
        pltpu.VMEM((NSLOT, BLK, d), _F32),
        pltpu.VMEM_SHARED((N_PAD, d), _F32),
        pltpu.SemaphoreType.DMA((NSLOT,)),
        pltpu.SemaphoreType.DMA((NSLOT,)),
    ]

    @functools.partial(
        pl.kernel,
        out_type=out_type,
        mesh=_MESH,
        scratch_types=tuple(scratch),
        compiler_params=pltpu.CompilerParams(use_tc_tiling_on_sc=False),
    )
    def k(y_hbm, e_hbm, zD_hbm, out_hbm, src_v, dst_v, buf_v, acc_sh,
          gsem, ssem):
        c = lax.axis_index("c")
        s = lax.axis_index("s")
        wid = s * NC + c
        r0 = s * ROWS_PER_SUB
        pltpu.sync_copy(zD_hbm.at[pl.ds(r0, ROWS_PER_SUB)],
                        acc_sh.at[pl.ds(r0, ROWS_PER_SUB)])
        pltpu.sync_copy(e_hbm.at[wid], src_v)
        pltpu.sync_copy(e_hbm.at[NW + wid], dst_v)
        plsc.subcore_barrier()

        def wait_scatter(j, b):
            pltpu.make_async_copy(buf_v.at[j],
                                  acc_sh.at[dst_v.at[b]], ssem.at[j]).wait()

        def visit(b, j, first):
            pltpu.make_async_copy(y_hbm.at[src_v.at[b]], buf_v.at[j],
                                  gsem.at[j]).wait()
            jp = (j - 1) % NSLOT
            if not first:
                wait_scatter(jp, b)
            pltpu.async_copy(buf_v.at[j], acc_sh.at[dst_v.at[b]],
                             ssem.at[j], add=True)
            jf = (j + LOOKAHEAD) % NSLOT
            f = b + LOOKAHEAD

            @pl.when(f < NB)
            def _():
                pltpu.async_copy(y_hbm.at[src_v.at[f]], buf_v.at[jf],
                                 gsem.at[jf])

        for b0 in range(LOOKAHEAD):
            pltpu.async_copy(y_hbm.at[src_v.at[b0]], buf_v.at[b0],
                             gsem.at[b0])

        @pl.loop(0, NB_MAIN // NSLOT)
        def _(t):
            for j in range(NSLOT):
                b = t * NSLOT + j
                if j == 0:
                    @pl.when(t > 0)
                    def _():
                        visit(b, j, False)

                    @pl.when(t == 0)
                    def _():
                        visit(b, j, True)
                else:
                    visit(b, j, False)

        for b in range(NB_MAIN, NB):
            visit(b, b % NSLOT, False)

        wait_scatter((NB - 1) % NSLOT, 0)
        plsc.subcore_barrier()
        pltpu.sync_copy(acc_sh.at[pl.ds(r0, ROWS_PER_SUB)],
                        out_hbm.at[c, pl.ds(r0, ROWS_PER_SUB)])

    return k(y, e_r, zD)


def _sc_deg(e_r):
    @functools.partial(
        pl.kernel,
        out_type=jax.ShapeDtypeStruct((NC, N_PAD, 16), _F32),
        mesh=_MESH,
        scratch_types=(
            pltpu.VMEM((NB, BLK), jnp.int32),
            pltpu.VMEM((BLK, 16), _F32),
            pltpu.VMEM_SHARED((N_PAD, 16), _F32),
            pltpu.SemaphoreType.DMA((2,)),
        ),
        compiler_params=pltpu.CompilerParams(use_tc_tiling_on_sc=False),
    )
    def k(e_hbm, ones_hbm, z16_hbm, deg_hbm, dst_v, ones_v, deg_sh, dsem):
        c = lax.axis_index("c")
        s = lax.axis_index("s")
        wid = s * NC + c
        r0 = s * ROWS_PER_SUB
        pltpu.sync_copy(z16_hbm.at[pl.ds(r0, ROWS_PER_SUB)],
                        deg_sh.at[pl.ds(r0, ROWS_PER_SUB)])
        pltpu.sync_copy(ones_hbm, ones_v)
        pltpu.sync_copy(e_hbm.at[NW + wid], dst_v)
        plsc.subcore_barrier()

        def dwait(j, b):
            pltpu.make_async_copy(ones_v, deg_sh.at[dst_v.at[b]],
                                  dsem.at[j]).wait()

        @pl.loop(0, NB // 2)
        def _(t):
            for j in range(2):
                b = t * 2 + j
                if j == 0:
                    @pl.when(t > 0)
                    def _():
                        dwait(1, b)
                else:
                    dwait(0, b)
                pltpu.async_copy(ones_v, deg_sh.at[dst_v.at[b]],
                                 dsem.at[j], add=True)

        dwait((NB - 1) % 2, 0)
        plsc.subcore_barrier()
        pltpu.sync_copy(deg_sh.at[pl.ds(r0, ROWS_PER_SUB)],
                        deg_hbm.at[c, pl.ds(r0, ROWS_PER_SUB)])

    return k(e_r, _ONES_BLK, _Z16)


def _dot(a, b):
    return lax.dot_general(a, b, (((1,), (0,)), ((), ())),
                           preferred_element_type=_F32,
                           precision=lax.Precision.HIGHEST)


RG = 2
RB = N_NODES // RG

_TC_PARAMS = pltpu.CompilerParams(dimension_semantics=("parallel",))


def _tc_y1(x, W_l):
    def body(x_ref, wl_ref, y_ref):
        y_ref[...] = _dot(x_ref[...], wl_ref[...])

    return pl.pallas_call(
        body,
        grid=(RG,),
        in_specs=[pl.BlockSpec((RB, D_IN), lambda i: (i, 0)),
                  pl.BlockSpec((D_IN, D_H1), lambda i: (0, 0))],
        out_specs=pl.BlockSpec((RB, D_H1), lambda i: (i, 0)),
        out_shape=jax.ShapeDtypeStruct((N_NODES, D_H1), _F32),
        compiler_params=_TC_PARAMS,
    )(x, W_l)


def _tc_root(x, W_r, b, d):
    din = x.shape[1]

    def body(x_ref, wr_ref, b_ref, r_ref):
        r_ref[...] = _dot(x_ref[...], wr_ref[...]) + b_ref[...]

    return pl.pallas_call(
        body,
        grid=(RG,),
        in_specs=[pl.BlockSpec((RB, din), lambda i: (i, 0)),
                  pl.BlockSpec((din, d), lambda i: (0, 0)),
                  pl.BlockSpec((1, d), lambda i: (0, 0))],
        out_specs=pl.BlockSpec((RB, d), lambda i: (i, 0)),
        out_shape=jax.ShapeDtypeStruct((N_NODES, d), _F32),
        compiler_params=_TC_PARAMS,
    )(x, W_r, b)


def _tc_mid(p1, pdeg, r1, W2_l):
    def body(p1_ref, pdeg_ref, r1_ref, wl_ref, h_ref, y2_ref):
        deg = pdeg_ref[0, :, 0:1] + pdeg_ref[1, :, 0:1]
        degc = jnp.maximum(deg, 1.0)
        h = jnp.maximum((p1_ref[0] + p1_ref[1]) / degc + r1_ref[...], 0.0)
        h_ref[...] = h
        y2_ref[...] = _dot(h, wl_ref[...])

    return pl.pallas_call(
        body,
        grid=(RG,),
        in_specs=[pl.BlockSpec((NC, RB, D_H1), lambda i: (0, i, 0)),
                  pl.BlockSpec((NC, RB, 16), lambda i: (0, i, 0)),
                  pl.BlockSpec((RB, D_H1), lambda i: (i, 0)),
                  pl.BlockSpec((D_H1, D_H2), lambda i: (0, 0))],
        out_specs=(pl.BlockSpec((RB, D_H1), lambda i: (i, 0)),
                   pl.BlockSpec((RB, D_H2), lambda i: (i, 0))),
        out_shape=(jax.ShapeDtypeStruct((N_NODES, D_H1), _F32),
                   jax.ShapeDtypeStruct((N_NODES, D_H2), _F32)),
        compiler_params=_TC_PARAMS,
    )(p1, pdeg, r1, W2_l)


def _tc_final(p2, pdeg, r2):
    def body(p2_ref, pdeg_ref, r2_ref, o_ref):
        deg = pdeg_ref[0, :, 0:1] + pdeg_ref[1, :, 0:1]
        degc = jnp.maximum(deg, 1.0)
        o_ref[...] = (p2_ref[0] + p2_ref[1]) / degc + r2_ref[...]

    return pl.pallas_call(
        body,
        grid=(RG,),
        in_specs=[pl.BlockSpec((NC, RB, D_H2), lambda i: (0, i, 0)),
                  pl.BlockSpec((NC, RB, 16), lambda i: (0, i, 0)),
                  pl.BlockSpec((RB, D_H2), lambda i: (i, 0))],
        out_specs=pl.BlockSpec((RB, D_H2), lambda i: (i, 0)),
        out_shape=jax.ShapeDtypeStruct((N_NODES, D_H2), _F32),
        compiler_params=_TC_PARAMS,
    )(p2, pdeg, r2)


def kernel(x, edge_index, W1_l, W1_r, b1, W2_l, W2_r, b2):
    e_r = edge_index.reshape(2 * NW, NB, BLK)

    pdeg = _sc_deg(e_r)
    y1 = _tc_y1(x, W1_l)
    r1 = _tc_root(x, W1_r, b1.reshape(1, D_H1), D_H1)
    p1 = _sc_segsum(y1, e_r)
    h, y2 = _tc_mid(p1, pdeg, r1, W2_l)
    r2 = _tc_root(h, W2_r, b2.reshape(1, D_H2), D_H2)
    p2 = _sc_segsum(y2, e_r)
    return _tc_final(p2, pdeg, r2)


# speedup vs baseline: 1.0178x; 1.0178x over previous
import functools

import jax
import jax.numpy as jnp
import numpy as np
from jax import lax
from jax.experimental import pallas as pl
from jax.experimental.pallas import tpu as pltpu
from jax.experimental.pallas import tpu_sc as plsc

N_NODES = 10000
N_EDGES = 320000
D_IN = 128
D_H1 = 64
D_H2 = 32

NC = 2
NS = 16
NW = NC * NS
E_PER_W = N_EDGES // NW
BLK = 200
NB = E_PER_W // BLK
ROWS_PER_SUB = 632
N_PAD = ROWS_PER_SUB * NS
NSLOT = 4
LOOKAHEAD = 3

_MESH = plsc.VectorSubcoreMesh(core_axis_name="c", subcore_axis_name="s")

_F32 = jnp.float32

_ONES_BLK = np.ones((BLK, 16), np.float32)
NB_MAIN = (NB // NSLOT) * NSLOT

_Z64 = np.zeros((N_PAD, D_H1), np.float32)
_Z32 = np.zeros((N_PAD, D_H2), np.float32)
_Z16 = np.zeros((N_PAD, 16), np.float32)


def _sc_segsum(y, e_r):
    d = y.shape[1]
    zD = _Z64 if d == D_H1 else _Z32
    out_type = jax.ShapeDtypeStruct((NC, N_PAD, d), _F32)
    scratch = [
        pltpu.VMEM((NB, BLK), jnp.int32),
        pltpu.VMEM((NB, BLK), jnp.int32),
        pltpu.VMEM((NSLOT, BLK, d), _F32),
        pltpu.VMEM_SHARED((N_PAD, d), _F32),
        pltpu.SemaphoreType.DMA((NSLOT,)),
        pltpu.SemaphoreType.DMA((NSLOT,)),
    ]

    @functools.partial(
        pl.kernel,
        out_type=out_type,
        mesh=_MESH,
        scratch_types=tuple(scratch),
        compiler_params=pltpu.CompilerParams(use_tc_tiling_on_sc=False),
    )
    def k(y_hbm, e_hbm, zD_hbm, out_hbm, src_v, dst_v, buf_v, acc_sh,
          gsem, ssem):
        c = lax.axis_index("c")
        s = lax.axis_index("s")
        wid = s * NC + c
        r0 = s * ROWS_PER_SUB
        pltpu.sync_copy(zD_hbm.at[pl.ds(r0, ROWS_PER_SUB)],
                        acc_sh.at[pl.ds(r0, ROWS_PER_SUB)])
        pltpu.sync_copy(e_hbm.at[wid], src_v)
        pltpu.sync_copy(e_hbm.at[NW + wid], dst_v)
        plsc.subcore_barrier()

        def wait_scatter(j, b):
            pltpu.make_async_copy(buf_v.at[j],
                                  acc_sh.at[dst_v.at[b]], ssem.at[j]).wait()

        def visit(b, j, first):
            pltpu.make_async_copy(y_hbm.at[src_v.at[b]], buf_v.at[j],
                                  gsem.at[j]).wait()
            jp = (j - 1) % NSLOT
            if not first:
                wait_scatter(jp, b)
            pltpu.async_copy(buf_v.at[j], acc_sh.at[dst_v.at[b]],
                             ssem.at[j], add=True)
            jf = (j + LOOKAHEAD) % NSLOT
            f = b + LOOKAHEAD

            @pl.when(f < NB)
            def _():
                pltpu.async_copy(y_hbm.at[src_v.at[f]], buf_v.at[jf],
                                 gsem.at[jf])

        for b0 in range(LOOKAHEAD):
            pltpu.async_copy(y_hbm.at[src_v.at[b0]], buf_v.at[b0],
                             gsem.at[b0])

        @pl.loop(0, NB_MAIN // NSLOT)
        def _(t):
            for j in range(NSLOT):
                b = t * NSLOT + j
                if j == 0:
                    @pl.when(t > 0)
                    def _():
                        visit(b, j, False)

                    @pl.when(t == 0)
                    def _():
                        visit(b, j, True)
                else:
                    visit(b, j, False)

        for b in range(NB_MAIN, NB):
            visit(b, b % NSLOT, False)

        wait_scatter((NB - 1) % NSLOT, 0)
        plsc.subcore_barrier()
        pltpu.sync_copy(acc_sh.at[pl.ds(r0, ROWS_PER_SUB)],
                        out_hbm.at[c, pl.ds(r0, ROWS_PER_SUB)])

    return k(y, e_r, zD)


def _sc_deg(e_r):
    @functools.partial(
        pl.kernel,
        out_type=jax.ShapeDtypeStruct((NC, N_PAD, 16), _F32),
        mesh=_MESH,
        scratch_types=(
            pltpu.VMEM((NB, BLK), jnp.int32),
            pltpu.VMEM((BLK, 16), _F32),
            pltpu.VMEM_SHARED((N_PAD, 16), _F32),
            pltpu.SemaphoreType.DMA((2,)),
        ),
        compiler_params=pltpu.CompilerParams(use_tc_tiling_on_sc=False),
    )
    def k(e_hbm, ones_hbm, z16_hbm, deg_hbm, dst_v, ones_v, deg_sh, dsem):
        c = lax.axis_index("c")
        s = lax.axis_index("s")
        wid = s * NC + c
        r0 = s * ROWS_PER_SUB
        pltpu.sync_copy(z16_hbm.at[pl.ds(r0, ROWS_PER_SUB)],
                        deg_sh.at[pl.ds(r0, ROWS_PER_SUB)])
        pltpu.sync_copy(ones_hbm, ones_v)
        pltpu.sync_copy(e_hbm.at[NW + wid], dst_v)
        plsc.subcore_barrier()

        def dwait(j, b):
            pltpu.make_async_copy(ones_v, deg_sh.at[dst_v.at[b]],
                                  dsem.at[j]).wait()

        @pl.loop(0, NB // 2)
        def _(t):
            for j in range(2):
                b = t * 2 + j
                if j == 0:
                    @pl.when(t > 0)
                    def _():
                        dwait(1, b)
                else:
                    dwait(0, b)
                pltpu.async_copy(ones_v, deg_sh.at[dst_v.at[b]],
                                 dsem.at[j], add=True)

        dwait((NB - 1) % 2, 0)
        plsc.subcore_barrier()
        pltpu.sync_copy(deg_sh.at[pl.ds(r0, ROWS_PER_SUB)],
                        deg_hbm.at[c, pl.ds(r0, ROWS_PER_SUB)])

    return k(e_r, _ONES_BLK, _Z16)


def _dot(a, b):
    return lax.dot_general(a, b, (((1,), (0,)), ((), ())),
                           preferred_element_type=_F32,
                           precision=lax.Precision.HIGHEST)


RG = 5
RB = N_NODES // RG

_TC_PARAMS = pltpu.CompilerParams(dimension_semantics=("parallel",))


def _tc_y1(x, W_l):
    def body(x_ref, wl_ref, y_ref):
        y_ref[...] = _dot(x_ref[...], wl_ref[...])

    return pl.pallas_call(
        body,
        grid=(RG,),
        in_specs=[pl.BlockSpec((RB, D_IN), lambda i: (i, 0)),
                  pl.BlockSpec((D_IN, D_H1), lambda i: (0, 0))],
        out_specs=pl.BlockSpec((RB, D_H1), lambda i: (i, 0)),
        out_shape=jax.ShapeDtypeStruct((N_NODES, D_H1), _F32),
        compiler_params=_TC_PARAMS,
    )(x, W_l)


def _tc_root(x, W_r, b, d):
    din = x.shape[1]

    def body(x_ref, wr_ref, b_ref, r_ref):
        r_ref[...] = _dot(x_ref[...], wr_ref[...]) + b_ref[...]

    return pl.pallas_call(
        body,
        grid=(RG,),
        in_specs=[pl.BlockSpec((RB, din), lambda i: (i, 0)),
                  pl.BlockSpec((din, d), lambda i: (0, 0)),
                  pl.BlockSpec((1, d), lambda i: (0, 0))],
        out_specs=pl.BlockSpec((RB, d), lambda i: (i, 0)),
        out_shape=jax.ShapeDtypeStruct((N_NODES, d), _F32),
        compiler_params=_TC_PARAMS,
    )(x, W_r, b)


def _tc_mid(p1, pdeg, r1, W2_l):
    def body(p1_ref, pdeg_ref, r1_ref, wl_ref, h_ref, y2_ref):
        deg = pdeg_ref[0, :, 0:1] + pdeg_ref[1, :, 0:1]
        degc = jnp.maximum(deg, 1.0)
        h = jnp.maximum((p1_ref[0] + p1_ref[1]) / degc + r1_ref[...], 0.0)
        h_ref[...] = h
        y2_ref[...] = _dot(h, wl_ref[...])

    return pl.pallas_call(
        body,
        grid=(RG,),
        in_specs=[pl.BlockSpec((NC, RB, D_H1), lambda i: (0, i, 0)),
                  pl.BlockSpec((NC, RB, 16), lambda i: (0, i, 0)),
                  pl.BlockSpec((RB, D_H1), lambda i: (i, 0)),
                  pl.BlockSpec((D_H1, D_H2), lambda i: (0, 0))],
        out_specs=(pl.BlockSpec((RB, D_H1), lambda i: (i, 0)),
                   pl.BlockSpec((RB, D_H2), lambda i: (i, 0))),
        out_shape=(jax.ShapeDtypeStruct((N_NODES, D_H1), _F32),
                   jax.ShapeDtypeStruct((N_NODES, D_H2), _F32)),
        compiler_params=_TC_PARAMS,
    )(p1, pdeg, r1, W2_l)


def _tc_final(p2, pdeg, r2):
    def body(p2_ref, pdeg_ref, r2_ref, o_ref):
        deg = pdeg_ref[0, :, 0:1] + pdeg_ref[1, :, 0:1]
        degc = jnp.maximum(deg, 1.0)
        o_ref[...] = (p2_ref[0] + p2_ref[1]) / degc + r2_ref[...]

    return pl.pallas_call(
        body,
        grid=(RG,),
        in_specs=[pl.BlockSpec((NC, RB, D_H2), lambda i: (0, i, 0)),
                  pl.BlockSpec((NC, RB, 16), lambda i: (0, i, 0)),
                  pl.BlockSpec((RB, D_H2), lambda i: (i, 0))],
        out_specs=pl.BlockSpec((RB, D_H2), lambda i: (i, 0)),
        out_shape=jax.ShapeDtypeStruct((N_NODES, D_H2), _F32),
        compiler_params=_TC_PARAMS,
    )(p2, pdeg, r2)


def kernel(x, edge_index, W1_l, W1_r, b1, W2_l, W2_r, b2):
    e_r = edge_index.reshape(2 * NW, NB, BLK)

    pdeg = _sc_deg(e_r)
    y1 = _tc_y1(x, W1_l)
    r1 = _tc_root(x, W1_r, b1.reshape(1, D_H1), D_H1)
    p1 = _sc_segsum(y1, e_r)
    h, y2 = _tc_mid(p1, pdeg, r1, W2_l)
    r2 = _tc_root(h, W2_r, b2.reshape(1, D_H2), D_H2)
    p2 = _sc_segsum(y2, e_r)
    return _tc_final(p2, pdeg, r2)
